# baseline (speedup 1.0000x reference)
import functools

import jax
import jax.numpy as jnp
from jax import lax
from jax.experimental import pallas as pl
from jax.experimental.pallas import tpu as pltpu
from jax.experimental.pallas import tpu_sc as plsc

_B, _C, _N, _K = 8, 64, 16384, 64
_D_VAR, _D_DIST = 0.5, 1.5
_PAR_VAR, _PAR_DIST, _PAR_REG = 1.0, 1.0, 0.001

_V = 16
_SPLITS = (5, 3)
_NW = 32
_CH = 128
_NB = 3


def _prep_body(pred_ref, pt_ref, vc_ref):
    x = pred_ref[0]
    m = jnp.sum(x, axis=0) * (1.0 / _C)
    d = x - m[None, :]
    nrm = jnp.sqrt(jnp.sum(d * d, axis=0))
    v = jnp.maximum(nrm - _D_VAR, 0.0)
    v = v * v
    pt_ref[0] = x.T
    col = lax.broadcasted_iota(jnp.int32, (_N, _V), 1)
    vc_ref[0] = jnp.where(col == 0, v[:, None],
                          jnp.where(col == 1, 1.0, 0.0))


def _prep_part(prediction, b0, nb):
    return pl.pallas_call(
        _prep_body,
        grid=(nb, 1),
        in_specs=[pl.BlockSpec((1, _C, _N),
                               lambda b, i, b0=b0: (b + b0, 0, i))],
        out_specs=[
            pl.BlockSpec((1, _N, _C), lambda b, i: (b, i, 0)),
            pl.BlockSpec((1, _N, _V), lambda b, i: (b, i, 0)),
        ],
        out_shape=[
            jax.ShapeDtypeStruct((nb, _N, _C), jnp.float32),
            jax.ShapeDtypeStruct((nb, _N, _V), jnp.float32),
        ],
    )(prediction)


def _make_segsum_body(nb):
    ppw = nb * _N // _NW
    nch = ppw // _CH
    ar = nb * _K
    nz = ar // 32

    def _segsum_body(pt, vc, lab2, outa, outb,
                     buf0, buf1, buf2, vbuf0, vbuf1, vbuf2, idx_all,
                     acca, accb,
                     gs0, gs1, gs2, as0, as1, as2,
                     hs0, hs1, hs2, bs0, bs1, bs2):
        bufs, vbufs = [buf0, buf1, buf2], [vbuf0, vbuf1, vbuf2]
        gsem, asem = [gs0, gs1, gs2], [as0, as1, as2]
        hsem, bsem = [hs0, hs1, hs2], [bs0, bs1, bs2]
        cid = lax.axis_index("c")
        sid = lax.axis_index("s")
        wid = sid * 2 + cid

        zvec = jnp.zeros((16,), jnp.float32)

        def zero_row(r, carry):
            for j in range(_C // 16):
                buf0[r, pl.ds(j * 16, 16)] = zvec
            vbuf0[r, :] = zvec
            return carry

        lax.fori_loop(0, 32, zero_row, 0)

        @pl.when(sid < nz)
        def _zero_acc():
            pltpu.sync_copy(buf0.at[pl.ds(0, 32)],
                            acca.at[pl.ds(sid * 32, 32)])
            pltpu.sync_copy(vbuf0.at[pl.ds(0, 32)],
                            accb.at[pl.ds(sid * 32, 32)])

        plsc.subcore_barrier()

        pltpu.sync_copy(lab2.at[wid], idx_all)

        base = wid * ppw

        def gdesc(i):
            return pltpu.make_async_copy(
                pt.at[pl.ds(base + i * _CH, _CH)], bufs[i % _NB],
                gsem[i % _NB])

        def hdesc(i):
            return pltpu.make_async_copy(
                vc.at[pl.ds(base + i * _CH, _CH)], vbufs[i % _NB],
                hsem[i % _NB])

        def adesc(i):
            return pltpu.make_async_copy(
                bufs[i % _NB], acca.at[idx_all.at[i]], asem[i % _NB])

        def bdesc(i):
            return pltpu.make_async_copy(
                vbufs[i % _NB], accb.at[idx_all.at[i]], bsem[i % _NB])

        for i in range(_NB - 1):
            gdesc(i).start()
            hdesc(i).start()
        for i in range(nch):
            if i + _NB - 1 < nch:
                if i - 1 >= 0:
                    adesc(i - 1).wait()
                    bdesc(i - 1).wait()
                gdesc(i + _NB - 1).start()
                hdesc(i + _NB - 1).start()
            gdesc(i).wait()
            hdesc(i).wait()
            adesc(i).start(add=True)
            bdesc(i).start(add=True)
        for i in range(nch - _NB, nch):
            adesc(i).wait()
            bdesc(i).wait()
        plsc.subcore_barrier()

        @pl.when(sid < nz)
        def _write_out():
            pltpu.sync_copy(acca.at[pl.ds(sid * 32, 32)],
                            outa.at[cid, pl.ds(sid * 32, 32)])
            pltpu.sync_copy(accb.at[pl.ds(sid * 32, 32)],
                            outb.at[cid, pl.ds(sid * 32, 32)])

    return _segsum_body


@functools.cache
def _segsum(nb):
    nch = nb * _N // _NW // _CH
    ar = nb * _K
    return pl.kernel(
        _make_segsum_body(nb),
        out_type=[
            jax.ShapeDtypeStruct((2, ar, _C), jnp.float32),
            jax.ShapeDtypeStruct((2, ar, _V), jnp.float32),
        ],
        mesh=plsc.VectorSubcoreMesh(core_axis_name="c", subcore_axis_name="s"),
        scratch_types=[
            pltpu.VMEM((_CH, _C), jnp.float32),
            pltpu.VMEM((_CH, _C), jnp.float32),
            pltpu.VMEM((_CH, _C), jnp.float32),
            pltpu.VMEM((_CH, _V), jnp.float32),
            pltpu.VMEM((_CH, _V), jnp.float32),
            pltpu.VMEM((_CH, _V), jnp.float32),
            pltpu.VMEM((nch, _CH), jnp.int32),
            pltpu.VMEM_SHARED((ar, _C), jnp.float32),
            pltpu.VMEM_SHARED((ar, _V), jnp.float32),
            pltpu.SemaphoreType.DMA, pltpu.SemaphoreType.DMA,
            pltpu.SemaphoreType.DMA, pltpu.SemaphoreType.DMA,
            pltpu.SemaphoreType.DMA, pltpu.SemaphoreType.DMA,
            pltpu.SemaphoreType.DMA, pltpu.SemaphoreType.DMA,
            pltpu.SemaphoreType.DMA, pltpu.SemaphoreType.DMA,
            pltpu.SemaphoreType.DMA, pltpu.SemaphoreType.DMA,
        ],
    )


def _finish_body(*refs):
    p_refs, out_ref = refs[:-1], refs[-1]
    total = jnp.float32(0.0)
    for pa_ref, pb_ref in zip(p_refs[0::2], p_refs[1::2]):
        a = pa_ref[0] + pa_ref[1]
        e = pb_ref[0] + pb_ref[1]
        nb = a.shape[0] // _K
        for b in range(nb):
            sums = a[b * _K:(b + 1) * _K, :]
            blk = e[b * _K:(b + 1) * _K, :]
            segv = blk[:, 0:1]
            counts = blk[:, 1:2]
            center = sums / counts
            sq = jnp.sum(center * center, axis=0)
            l_reg = jnp.sum(jnp.sqrt(sq)) / _K
            var_b = jnp.sum(segv / counts) / _K
            g = lax.dot_general(center, center, (((0,), (0,)), ((), ())),
                                preferred_element_type=jnp.float32,
                                precision=lax.Precision.HIGHEST)
            dist = -2.0 * g + sq[:, None] + sq[None, :]
            dm = jnp.sqrt(jnp.maximum(dist, 0.0))
            hinge = jnp.maximum(2.0 * _D_DIST - dm, 0.0)
            dist_b = jnp.sum(hinge * hinge) / (2.0 * _K * (_K - 1.0 + 1e-16))
            total = total + (_PAR_VAR * var_b + _PAR_DIST * dist_b
                             + _PAR_REG * l_reg)
    out_ref[...] = jnp.reshape(total, (1, 1))


def _finish(parts):
    return pl.pallas_call(
        _finish_body,
        out_shape=jax.ShapeDtypeStruct((1, 1), jnp.float32),
    )(*parts)


def kernel(prediction, label):
    lab = label.astype(jnp.int32)
    parts = []
    b0 = 0
    for nb in _SPLITS:
        offs = (jnp.arange(nb, dtype=jnp.int32) * _K)[:, None]
        nch = nb * _N // _NW // _CH
        labs = (lab[b0:b0 + nb] + offs).reshape(_NW, nch, _CH)
        pt, vc = _prep_part(prediction, b0, nb)
        pa, pb = _segsum(nb)(pt.reshape(nb * _N, _C),
                             vc.reshape(nb * _N, _V), labs)
        parts.extend([pa, pb])
        b0 += nb
    return _finish(parts)[0, 0]


# speedup vs baseline: 1.2726x; 1.2726x over previous
import functools

import jax
import jax.numpy as jnp
from jax import lax
from jax.experimental import pallas as pl
from jax.experimental.pallas import tpu as pltpu
from jax.experimental.pallas import tpu_sc as plsc

_B, _C, _N, _K = 8, 64, 16384, 64
_D_VAR, _D_DIST = 0.5, 1.5
_PAR_VAR, _PAR_DIST, _PAR_REG = 1.0, 1.0, 0.001

_W = 80
_SPLITS = (5, 3)
_NW = 32
_CH = 128
_NB = 3


def _prep_body(pred_ref, out_ref):
    x = pred_ref[0]
    m = jnp.sum(x, axis=0) * (1.0 / _C)
    d = x - m[None, :]
    nrm = jnp.sqrt(jnp.sum(d * d, axis=0))
    v = jnp.maximum(nrm - _D_VAR, 0.0)
    v = v * v
    out_ref[0, :, 0:_C] = x.T
    col = lax.broadcasted_iota(jnp.int32, (_N, 16), 1)
    extra = jnp.where(col == 0, v[:, None],
                      jnp.where(col == 1, 1.0, 0.0))
    out_ref[0, :, _C:_W] = extra


def _prep_part(prediction, b0, nb):
    return pl.pallas_call(
        _prep_body,
        grid=(nb, 1),
        in_specs=[pl.BlockSpec((1, _C, _N),
                               lambda b, i, b0=b0: (b + b0, 0, i))],
        out_specs=pl.BlockSpec((1, _N, _W), lambda b, i: (b, i, 0)),
        out_shape=jax.ShapeDtypeStruct((nb, _N, _W), jnp.float32),
    )(prediction)


def _make_segsum_body(nb):
    ppw = nb * _N // _NW
    nch = ppw // _CH
    ar = nb * _K
    nz = ar // 32

    def _segsum_body(pred_ext, lab2, out, buf0, buf1, buf2, idx_all, acc,
                     gs0, gs1, gs2, as0, as1, as2):
        bufs = [buf0, buf1, buf2]
        gsem, asem = [gs0, gs1, gs2], [as0, as1, as2]
        cid = lax.axis_index("c")
        sid = lax.axis_index("s")
        wid = sid * 2 + cid

        zvec = jnp.zeros((16,), jnp.float32)

        def zero_row(r, carry):
            for j in range(_W // 16):
                buf0[r, pl.ds(j * 16, 16)] = zvec
            return carry

        lax.fori_loop(0, 32, zero_row, 0)

        @pl.when(sid < nz)
        def _zero_acc():
            pltpu.sync_copy(buf0.at[pl.ds(0, 32)],
                            acc.at[pl.ds(sid * 32, 32)])

        plsc.subcore_barrier()

        pltpu.sync_copy(lab2.at[wid], idx_all)

        base = wid * ppw

        def gdesc(i):
            return pltpu.make_async_copy(
                pred_ext.at[pl.ds(base + i * _CH, _CH)], bufs[i % _NB],
                gsem[i % _NB])

        def adesc(i):
            return pltpu.make_async_copy(
                bufs[i % _NB], acc.at[idx_all.at[i]], asem[i % _NB])

        for i in range(_NB - 1):
            gdesc(i).start()
        for i in range(nch):
            if i + _NB - 1 < nch:
                if i - 1 >= 0:
                    adesc(i - 1).wait()
                gdesc(i + _NB - 1).start()
            gdesc(i).wait()
            adesc(i).start(add=True)
        for i in range(nch - _NB, nch):
            adesc(i).wait()
        plsc.subcore_barrier()

        @pl.when(sid < nz)
        def _write_out():
            pltpu.sync_copy(acc.at[pl.ds(sid * 32, 32)],
                            out.at[cid, pl.ds(sid * 32, 32)])

    return _segsum_body


@functools.cache
def _segsum(nb):
    ppw = nb * _N // _NW
    nch = ppw // _CH
    ar = nb * _K
    return pl.kernel(
        _make_segsum_body(nb),
        out_type=jax.ShapeDtypeStruct((2, ar, _W), jnp.float32),
        mesh=plsc.VectorSubcoreMesh(core_axis_name="c", subcore_axis_name="s"),
        scratch_types=[
            pltpu.VMEM((_CH, _W), jnp.float32),
            pltpu.VMEM((_CH, _W), jnp.float32),
            pltpu.VMEM((_CH, _W), jnp.float32),
            pltpu.VMEM((nch, _CH), jnp.int32),
            pltpu.VMEM_SHARED((ar, _W), jnp.float32),
            pltpu.SemaphoreType.DMA, pltpu.SemaphoreType.DMA,
            pltpu.SemaphoreType.DMA, pltpu.SemaphoreType.DMA,
            pltpu.SemaphoreType.DMA, pltpu.SemaphoreType.DMA,
        ],
    )


def _finish_body(*refs):
    p_refs, out_ref = refs[:-1], refs[-1]
    total = jnp.float32(0.0)
    for p_ref in p_refs:
        a = p_ref[0] + p_ref[1]
        nb = a.shape[0] // _K
        for b in range(nb):
            blk = a[b * _K:(b + 1) * _K, :]
            sums = blk[:, 0:_C]
            segv = blk[:, _C:_C + 1]
            counts = blk[:, _C + 1:_C + 2]
            center = sums / counts
            sq = jnp.sum(center * center, axis=0)
            l_reg = jnp.sum(jnp.sqrt(sq)) / _K
            var_b = jnp.sum(segv / counts) / _K
            g = lax.dot_general(center, center, (((0,), (0,)), ((), ())),
                                preferred_element_type=jnp.float32,
                                precision=lax.Precision.HIGHEST)
            dist = -2.0 * g + sq[:, None] + sq[None, :]
            dm = jnp.sqrt(jnp.maximum(dist, 0.0))
            hinge = jnp.maximum(2.0 * _D_DIST - dm, 0.0)
            dist_b = jnp.sum(hinge * hinge) / (2.0 * _K * (_K - 1.0 + 1e-16))
            total = total + (_PAR_VAR * var_b + _PAR_DIST * dist_b
                             + _PAR_REG * l_reg)
    out_ref[...] = jnp.reshape(total, (1, 1))


def _finish(parts):
    return pl.pallas_call(
        _finish_body,
        out_shape=jax.ShapeDtypeStruct((1, 1), jnp.float32),
    )(*parts)


def kernel(prediction, label):
    lab = label.astype(jnp.int32)
    parts = []
    b0 = 0
    for nb in _SPLITS:
        offs = (jnp.arange(nb, dtype=jnp.int32) * _K)[:, None]
        nch = nb * _N // _NW // _CH
        labs = (lab[b0:b0 + nb] + offs).reshape(_NW, nch, _CH)
        pes = _prep_part(prediction, b0, nb)
        parts.append(_segsum(nb)(pes.reshape(nb * _N, _W), labs))
        b0 += nb
    return _finish(parts)[0, 0]
